# baseline (speedup 1.0000x reference)
import functools

import jax
import jax.numpy as jnp
from jax import lax
from jax.experimental import pallas as pl
from jax.experimental.pallas import tpu as pltpu
from jax.experimental.pallas import tpu_sc as plsc

N = 100000
C = 1000
B = 4096
TAU = 0.3

_NC, _NS = 2, 16
_NW = _NC * _NS
_RPT = B // _NW
_G = 64


def _sc_gather_body(table_hbm, index_hbm, idxrp_hbm, t1_hbm, t2_hbm,
                    idx_all_v, idxrp_v, idx2_v, rows_v, sem):
    wid = lax.axis_index("s") * _NC + lax.axis_index("c")
    base = wid * _RPT
    pltpu.sync_copy(index_hbm, idx_all_v)
    pltpu.sync_copy(idxrp_hbm.at[pl.ds(base, _RPT)], idxrp_v)
    pltpu.async_copy(index_hbm.at[idxrp_v], idx2_v, sem).wait()
    for g in range(_RPT // _G):
        pltpu.async_copy(
            table_hbm.at[idx_all_v.at[pl.ds(base + g * _G, _G)]],
            rows_v, sem).wait()
        pltpu.sync_copy(rows_v, t1_hbm.at[pl.ds(base + g * _G, _G)])
    for g in range(_RPT // _G):
        pltpu.async_copy(
            table_hbm.at[idx2_v.at[pl.ds(g * _G, _G)]],
            rows_v, sem).wait()
        pltpu.sync_copy(rows_v, t2_hbm.at[pl.ds(base + g * _G, _G)])


def _sc_gather(table, index, idx_rp):
    run = pl.kernel(
        _sc_gather_body,
        mesh=plsc.VectorSubcoreMesh(core_axis_name="c", subcore_axis_name="s"),
        compiler_params=pltpu.CompilerParams(use_tc_tiling_on_sc=False),
        out_type=[jax.ShapeDtypeStruct((B, C), jnp.float32),
                  jax.ShapeDtypeStruct((B, C), jnp.float32)],
        scratch_types=[
            pltpu.VMEM((B,), jnp.int32),
            pltpu.VMEM((_RPT,), jnp.int32),
            pltpu.VMEM((_RPT,), jnp.int32),
            pltpu.VMEM((_G, C), jnp.float32),
            pltpu.SemaphoreType.DMA,
        ],
    )
    return run(table, index, idx_rp)


_BLK = 256
_GRID = B // _BLK


def _tc_body(cls_ref, q1_ref, q2_ref, t1_ref, t2_ref, out_ref):
    i = pl.program_id(0)
    x = cls_ref[...]
    m = jnp.max(x, axis=1, keepdims=True)
    ls = (x - m) - jnp.log(jnp.sum(jnp.exp(x - m), axis=1, keepdims=True))
    y1 = q1_ref[...] * (1.0 / TAU)
    m1 = jnp.max(y1, axis=1, keepdims=True)
    lq1 = (y1 - m1) - jnp.log(jnp.sum(jnp.exp(y1 - m1), axis=1, keepdims=True))
    y2 = q2_ref[...] * (1.0 / TAU)
    m2 = jnp.max(y2, axis=1, keepdims=True)
    lq2 = (y2 - m2) - jnp.log(jnp.sum(jnp.exp(y2 - m2), axis=1, keepdims=True))
    q = lq1 + lq2
    t1 = t1_ref[...]
    t2 = t2_ref[...]
    lt1 = jnp.log(jnp.where(t1 > 0, t1, 1.0))
    lt2 = jnp.log(jnp.where(t2 > 0, t2, 1.0))
    a = jnp.sum(t1 * ls)
    p = jnp.sum(t1 * q)
    r = jnp.sum(t2 * q)
    h1 = jnp.sum(t1 * lt1)
    h2 = jnp.sum(t2 * lt2)
    lane = lax.broadcasted_iota(jnp.int32, (1, 128), 1)
    vec = (jnp.where(lane == 0, a, 0.0) + jnp.where(lane == 1, p, 0.0)
           + jnp.where(lane == 2, r, 0.0) + jnp.where(lane == 3, h1, 0.0)
           + jnp.where(lane == 4, h2, 0.0))

    @pl.when(i == 0)
    def _():
        out_ref[...] = jnp.zeros_like(out_ref)

    out_ref[...] += vec


def _tc_reduce(cls_out1, lpm1, lpm2, t1, t2):
    return pl.pallas_call(
        _tc_body,
        grid=(_GRID,),
        in_specs=[pl.BlockSpec((_BLK, C), lambda i: (i, 0))] * 5,
        out_specs=pl.BlockSpec((1, 128), lambda i: (0, 0)),
        out_shape=jax.ShapeDtypeStruct((1, 128), jnp.float32),
    )(cls_out1, lpm1, lpm2, t1, t2)


def kernel(predicted_score_cls, cls_out1, cls_out2, logits_prot1,
           logits_prot2, logits_prot_1_mix, logits_prot_2_mix, idx_rp,
           Lambda, index):
    index = index.astype(jnp.int32)
    idx_rp = idx_rp.astype(jnp.int32)
    t1, t2 = _sc_gather(predicted_score_cls, index, idx_rp)
    scal = _tc_reduce(cls_out1, logits_prot_1_mix, logits_prot_2_mix, t1, t2)
    a, p, r, h1, h2 = scal[0, 0], scal[0, 1], scal[0, 2], scal[0, 3], scal[0, 4]
    bf = jnp.float32(B)
    lam = Lambda.astype(jnp.float32)
    cls_loss_1 = -a / bf
    sim_loss_2 = (2.0 * lam * h1 + 2.0 * (1.0 - lam) * h2
                  - lam * p - (1.0 - lam) * r) / bf
    return (cls_loss_1, sim_loss_2, jnp.float32(1.0))


# speedup vs baseline: 4.6418x; 4.6418x over previous
import jax
import jax.numpy as jnp
from jax import lax
from jax.experimental import pallas as pl
from jax.experimental.pallas import tpu as pltpu

N = 100000
C = 1000
B = 4096
TAU = 0.3

_BLK = 256
_GRID = B // _BLK


def _body(index_sm, idxrp_sm, table, cls_ref, q1_ref, q2_ref, out_ref,
          t1_buf, t2_buf, sem1, sem2):
    i = pl.program_id(0)

    def issue(b, _):
        gb = i * _BLK + b
        r1 = index_sm[gb]
        r2 = index_sm[idxrp_sm[gb]]
        pltpu.make_async_copy(
            table.at[pl.ds(r1, 1)], t1_buf.at[pl.ds(b, 1)], sem1).start()
        pltpu.make_async_copy(
            table.at[pl.ds(r2, 1)], t2_buf.at[pl.ds(b, 1)], sem2).start()
        return 0

    lax.fori_loop(0, _BLK, issue, 0, unroll=8)

    def drain(b, _):
        pltpu.make_async_copy(
            table.at[pl.ds(0, 1)], t1_buf.at[pl.ds(b, 1)], sem1).wait()
        pltpu.make_async_copy(
            table.at[pl.ds(0, 1)], t2_buf.at[pl.ds(b, 1)], sem2).wait()
        return 0

    lax.fori_loop(0, _BLK, drain, 0, unroll=8)

    x = cls_ref[...]
    m = jnp.max(x, axis=1, keepdims=True)
    ls = (x - m) - jnp.log(jnp.sum(jnp.exp(x - m), axis=1, keepdims=True))
    y1 = q1_ref[...] * (1.0 / TAU)
    m1 = jnp.max(y1, axis=1, keepdims=True)
    lq1 = (y1 - m1) - jnp.log(jnp.sum(jnp.exp(y1 - m1), axis=1, keepdims=True))
    y2 = q2_ref[...] * (1.0 / TAU)
    m2 = jnp.max(y2, axis=1, keepdims=True)
    lq2 = (y2 - m2) - jnp.log(jnp.sum(jnp.exp(y2 - m2), axis=1, keepdims=True))
    q = lq1 + lq2
    t1 = t1_buf[...]
    t2 = t2_buf[...]
    lt1 = jnp.log(jnp.where(t1 > 0, t1, 1.0))
    lt2 = jnp.log(jnp.where(t2 > 0, t2, 1.0))
    a = jnp.sum(t1 * ls)
    p = jnp.sum(t1 * q)
    r = jnp.sum(t2 * q)
    h1 = jnp.sum(t1 * lt1)
    h2 = jnp.sum(t2 * lt2)
    lane = lax.broadcasted_iota(jnp.int32, (1, 128), 1)
    vec = (jnp.where(lane == 0, a, 0.0) + jnp.where(lane == 1, p, 0.0)
           + jnp.where(lane == 2, r, 0.0) + jnp.where(lane == 3, h1, 0.0)
           + jnp.where(lane == 4, h2, 0.0))

    @pl.when(i == 0)
    def _():
        out_ref[...] = jnp.zeros_like(out_ref)

    out_ref[...] += vec


def _fused(index, idx_rp, table, cls_out1, lpm1, lpm2):
    grid_spec = pltpu.PrefetchScalarGridSpec(
        num_scalar_prefetch=2,
        grid=(_GRID,),
        in_specs=[
            pl.BlockSpec(memory_space=pltpu.MemorySpace.HBM),
            pl.BlockSpec((_BLK, C), lambda i, s1, s2: (i, 0)),
            pl.BlockSpec((_BLK, C), lambda i, s1, s2: (i, 0)),
            pl.BlockSpec((_BLK, C), lambda i, s1, s2: (i, 0)),
        ],
        out_specs=pl.BlockSpec((1, 128), lambda i, s1, s2: (0, 0)),
        scratch_shapes=[
            pltpu.VMEM((_BLK, C), jnp.float32),
            pltpu.VMEM((_BLK, C), jnp.float32),
            pltpu.SemaphoreType.DMA,
            pltpu.SemaphoreType.DMA,
        ],
    )
    return pl.pallas_call(
        _body,
        grid_spec=grid_spec,
        out_shape=jax.ShapeDtypeStruct((1, 128), jnp.float32),
    )(index, idx_rp, table, cls_out1, lpm1, lpm2)


def kernel(predicted_score_cls, cls_out1, cls_out2, logits_prot1,
           logits_prot2, logits_prot_1_mix, logits_prot_2_mix, idx_rp,
           Lambda, index):
    index = index.astype(jnp.int32)
    idx_rp = idx_rp.astype(jnp.int32)
    scal = _fused(index, idx_rp, predicted_score_cls, cls_out1,
                  logits_prot_1_mix, logits_prot_2_mix)
    a, p, r, h1, h2 = scal[0, 0], scal[0, 1], scal[0, 2], scal[0, 3], scal[0, 4]
    bf = jnp.float32(B)
    lam = Lambda.astype(jnp.float32)
    cls_loss_1 = -a / bf
    sim_loss_2 = (2.0 * lam * h1 + 2.0 * (1.0 - lam) * h2
                  - lam * p - (1.0 - lam) * r) / bf
    return (cls_loss_1, sim_loss_2, jnp.float32(1.0))


# speedup vs baseline: 4.6509x; 1.0019x over previous
import jax
import jax.numpy as jnp
from jax import lax
from jax.experimental import pallas as pl
from jax.experimental.pallas import tpu as pltpu

N = 100000
C = 1000
B = 4096
TAU = 0.3

_BLK = 256
_GRID = B // _BLK


def _body(index_sm, idxrp_sm, table, cls_ref, q1_ref, q2_ref, out_ref,
          t1_buf, t2_buf, sem1, sem2):
    i = pl.program_id(0)

    def issue(b, _):
        gb = i * _BLK + b
        r1 = index_sm[gb]
        r2 = index_sm[idxrp_sm[gb]]
        pltpu.make_async_copy(
            table.at[pl.ds(r1, 1)], t1_buf.at[pl.ds(b, 1)], sem1).start()
        pltpu.make_async_copy(
            table.at[pl.ds(r2, 1)], t2_buf.at[pl.ds(b, 1)], sem2).start()
        return 0

    lax.fori_loop(0, _BLK, issue, 0, unroll=8)

    pltpu.make_async_copy(table.at[pl.ds(0, _BLK)], t1_buf, sem1).wait()
    pltpu.make_async_copy(table.at[pl.ds(0, _BLK)], t2_buf, sem2).wait()

    x = cls_ref[...]
    m = jnp.max(x, axis=1, keepdims=True)
    ls = (x - m) - jnp.log(jnp.sum(jnp.exp(x - m), axis=1, keepdims=True))
    y1 = q1_ref[...] * (1.0 / TAU)
    m1 = jnp.max(y1, axis=1, keepdims=True)
    lq1 = (y1 - m1) - jnp.log(jnp.sum(jnp.exp(y1 - m1), axis=1, keepdims=True))
    y2 = q2_ref[...] * (1.0 / TAU)
    m2 = jnp.max(y2, axis=1, keepdims=True)
    lq2 = (y2 - m2) - jnp.log(jnp.sum(jnp.exp(y2 - m2), axis=1, keepdims=True))
    q = lq1 + lq2
    t1 = t1_buf[...]
    t2 = t2_buf[...]
    lt1 = jnp.log(jnp.where(t1 > 0, t1, 1.0))
    lt2 = jnp.log(jnp.where(t2 > 0, t2, 1.0))
    a = jnp.sum(t1 * ls)
    p = jnp.sum(t1 * q)
    r = jnp.sum(t2 * q)
    h1 = jnp.sum(t1 * lt1)
    h2 = jnp.sum(t2 * lt2)
    lane = lax.broadcasted_iota(jnp.int32, (1, 128), 1)
    vec = (jnp.where(lane == 0, a, 0.0) + jnp.where(lane == 1, p, 0.0)
           + jnp.where(lane == 2, r, 0.0) + jnp.where(lane == 3, h1, 0.0)
           + jnp.where(lane == 4, h2, 0.0))

    @pl.when(i == 0)
    def _():
        out_ref[...] = jnp.zeros_like(out_ref)

    out_ref[...] += vec


def _fused(index, idx_rp, table, cls_out1, lpm1, lpm2):
    grid_spec = pltpu.PrefetchScalarGridSpec(
        num_scalar_prefetch=2,
        grid=(_GRID,),
        in_specs=[
            pl.BlockSpec(memory_space=pltpu.MemorySpace.HBM),
            pl.BlockSpec((_BLK, C), lambda i, s1, s2: (i, 0)),
            pl.BlockSpec((_BLK, C), lambda i, s1, s2: (i, 0)),
            pl.BlockSpec((_BLK, C), lambda i, s1, s2: (i, 0)),
        ],
        out_specs=pl.BlockSpec((1, 128), lambda i, s1, s2: (0, 0)),
        scratch_shapes=[
            pltpu.VMEM((_BLK, C), jnp.float32),
            pltpu.VMEM((_BLK, C), jnp.float32),
            pltpu.SemaphoreType.DMA,
            pltpu.SemaphoreType.DMA,
        ],
    )
    return pl.pallas_call(
        _body,
        grid_spec=grid_spec,
        out_shape=jax.ShapeDtypeStruct((1, 128), jnp.float32),
    )(index, idx_rp, table, cls_out1, lpm1, lpm2)


def kernel(predicted_score_cls, cls_out1, cls_out2, logits_prot1,
           logits_prot2, logits_prot_1_mix, logits_prot_2_mix, idx_rp,
           Lambda, index):
    index = index.astype(jnp.int32)
    idx_rp = idx_rp.astype(jnp.int32)
    scal = _fused(index, idx_rp, predicted_score_cls, cls_out1,
                  logits_prot_1_mix, logits_prot_2_mix)
    a, p, r, h1, h2 = scal[0, 0], scal[0, 1], scal[0, 2], scal[0, 3], scal[0, 4]
    bf = jnp.float32(B)
    lam = Lambda.astype(jnp.float32)
    cls_loss_1 = -a / bf
    sim_loss_2 = (2.0 * lam * h1 + 2.0 * (1.0 - lam) * h2
                  - lam * p - (1.0 - lam) * r) / bf
    return (cls_loss_1, sim_loss_2, jnp.float32(1.0))
